# baseline (speedup 1.0000x reference)
import functools
import math

import jax
import jax.numpy as jnp
from jax.experimental import pallas as pl
from jax.experimental.pallas import tpu as pltpu

NH = 16


def _qkv_kernel(x_ref, w_ref, b_ref, o_ref, *, heads_per_step, hs):
    r = jnp.dot(x_ref[...], w_ref[...], preferred_element_type=jnp.float32)
    r = r + b_ref[...]
    for hh in range(heads_per_step):
        o_ref[0, hh] = r[:, hh * hs:(hh + 1) * hs]


def _attn_kernel(q_ref, k_ref, v_ref, o_ref, s_ref, *, bq, bk, scale):
    qi = pl.program_id(2)
    q = q_ref[0, 0]
    hs = q.shape[-1]
    nblk = qi + 1

    def p1(j, m_acc):
        kj = k_ref[0, 0, pl.ds(j * bk, bk), :]
        s = jax.lax.dot_general(q, kj, (((1,), (1,)), ((), ())),
                                preferred_element_type=jnp.float32) * scale
        s_ref[:, pl.ds(j * bk, bk)] = s
        return jnp.maximum(m_acc, s)

    m_acc = jnp.full((bq, bk), -1e30, dtype=jnp.float32)
    m_acc = jax.lax.fori_loop(0, qi, p1, m_acc)

    kd = k_ref[0, 0, pl.ds(qi * bk, bk), :]
    sd = jax.lax.dot_general(q, kd, (((1,), (1,)), ((), ())),
                             preferred_element_type=jnp.float32) * scale
    rows = jax.lax.broadcasted_iota(jnp.int32, (bq, bk), 0)
    cols = jax.lax.broadcasted_iota(jnp.int32, (bq, bk), 1)
    sd = jnp.where(cols <= rows, sd, -1e30)
    s_ref[:, pl.ds(qi * bk, bk)] = sd
    m_acc = jnp.maximum(m_acc, sd)
    m = jnp.max(m_acc, axis=1, keepdims=True)

    def p2(j, carry):
        l_acc, acc = carry
        s = s_ref[:, pl.ds(j * bk, bk)]
        p = jnp.exp(s - m)
        vj = v_ref[0, 0, pl.ds(j * bk, bk), :]
        acc = acc + jnp.dot(p, vj, preferred_element_type=jnp.float32)
        return l_acc + p, acc

    l0 = jnp.zeros((bq, bk), dtype=jnp.float32)
    a0 = jnp.zeros((bq, hs), dtype=jnp.float32)
    l_acc, acc = jax.lax.fori_loop(0, nblk, p2, (l0, a0))
    l = jnp.sum(l_acc, axis=1, keepdims=True)
    o_ref[0, 0] = acc / l


def _proj_kernel(y_ref, w_ref, b_ref, o_ref, *, nh, hs):
    acc = jnp.zeros((y_ref.shape[2], w_ref.shape[2]), dtype=jnp.float32)
    for h in range(nh):
        acc = acc + jnp.dot(y_ref[0, h], w_ref[h],
                            preferred_element_type=jnp.float32)
    o_ref[0] = acc + b_ref[...]


def kernel(x, W_attn, b_attn, W_proj, b_proj):
    B, T, C = x.shape
    HS = C // NH
    G = 3 * NH

    x2 = x.reshape(B * T, C)

    heads_per_step = 4
    bn1 = heads_per_step * HS
    ng1 = G // heads_per_step
    qkv = pl.pallas_call(
        functools.partial(_qkv_kernel, heads_per_step=heads_per_step, hs=HS),
        grid=(B, ng1),
        in_specs=[
            pl.BlockSpec((T, C), lambda b, j: (b, 0)),
            pl.BlockSpec((C, bn1), lambda b, j: (0, j)),
            pl.BlockSpec((1, bn1), lambda b, j: (0, j)),
        ],
        out_specs=pl.BlockSpec((1, heads_per_step, T, HS),
                               lambda b, j: (b, j, 0, 0)),
        out_shape=jax.ShapeDtypeStruct((B, G, T, HS), jnp.float32),
    )(x2, W_attn, b_attn.reshape(1, 3 * C))

    BQ = 256
    BK = 256
    nq = T // BQ
    scale = 1.0 / math.sqrt(HS)
    y = pl.pallas_call(
        functools.partial(_attn_kernel, bq=BQ, bk=BK, scale=scale),
        grid=(B, NH, nq),
        in_specs=[
            pl.BlockSpec((1, 1, BQ, HS), lambda b, h, qi: (b, h, qi, 0)),
            pl.BlockSpec((1, 1, T, HS), lambda b, h, qi: (b, NH + h, 0, 0)),
            pl.BlockSpec((1, 1, T, HS), lambda b, h, qi: (b, 2 * NH + h, 0, 0)),
        ],
        out_specs=pl.BlockSpec((1, 1, BQ, HS), lambda b, h, qi: (b, h, qi, 0)),
        out_shape=jax.ShapeDtypeStruct((B, NH, T, HS), jnp.float32),
        scratch_shapes=[pltpu.VMEM((BQ, T), jnp.float32)],
    )(qkv, qkv, qkv)

    bn3 = 512
    nn3 = C // bn3
    out = pl.pallas_call(
        functools.partial(_proj_kernel, nh=NH, hs=HS),
        grid=(B, nn3),
        in_specs=[
            pl.BlockSpec((1, NH, T, HS), lambda b, j: (b, 0, 0, 0)),
            pl.BlockSpec((NH, HS, bn3), lambda b, j: (0, 0, j)),
            pl.BlockSpec((1, bn3), lambda b, j: (0, j)),
        ],
        out_specs=pl.BlockSpec((1, T, bn3), lambda b, j: (b, 0, j)),
        out_shape=jax.ShapeDtypeStruct((B, T, C), jnp.float32),
    )(y, W_proj.reshape(NH, HS, C), b_proj.reshape(1, C))

    return out


# speedup vs baseline: 2.1755x; 2.1755x over previous
import functools
import math

import jax
import jax.numpy as jnp
from jax.experimental import pallas as pl
from jax.experimental.pallas import tpu as pltpu

NH = 16


def _qkv_kernel(x_ref, w_ref, b_ref, o_ref, *, heads_per_step, hs):
    r = jnp.dot(x_ref[...], w_ref[...], preferred_element_type=jnp.float32)
    r = r + b_ref[...]
    for hh in range(heads_per_step):
        o_ref[0, hh] = r[:, hh * hs:(hh + 1) * hs]


def _attn_kernel(q_ref, k_ref, v_ref, o_ref, *, bq, nq, scale):
    hs = q_ref.shape[3]
    for qi in range(nq):
        q = q_ref[0, 0, qi * bq:(qi + 1) * bq, :]
        s_blocks = []
        m_acc = None
        for j in range(qi + 1):
            kj = k_ref[0, 0, j * bq:(j + 1) * bq, :]
            s = jax.lax.dot_general(q, kj, (((1,), (1,)), ((), ())),
                                    preferred_element_type=jnp.float32) * scale
            if j == qi:
                rows = jax.lax.broadcasted_iota(jnp.int32, (bq, bq), 0)
                cols = jax.lax.broadcasted_iota(jnp.int32, (bq, bq), 1)
                s = jnp.where(cols <= rows, s, -1e30)
            s_blocks.append(s)
            m_acc = s if m_acc is None else jnp.maximum(m_acc, s)
        m = jnp.max(m_acc, axis=1, keepdims=True)
        l_acc = jnp.zeros((bq, bq), dtype=jnp.float32)
        acc = jnp.zeros((bq, hs), dtype=jnp.float32)
        for j in range(qi + 1):
            p = jnp.exp(s_blocks[j] - m)
            l_acc = l_acc + p
            vj = v_ref[0, 0, j * bq:(j + 1) * bq, :]
            acc = acc + jnp.dot(p, vj, preferred_element_type=jnp.float32)
        l = jnp.sum(l_acc, axis=1, keepdims=True)
        o_ref[0, 0, qi * bq:(qi + 1) * bq, :] = acc / l


def _proj_kernel(y_ref, w_ref, b_ref, o_ref, *, nh, hs):
    acc = jnp.zeros((y_ref.shape[2], w_ref.shape[2]), dtype=jnp.float32)
    for h in range(nh):
        acc = acc + jnp.dot(y_ref[0, h], w_ref[h],
                            preferred_element_type=jnp.float32)
    o_ref[0] = acc + b_ref[...]


def kernel(x, W_attn, b_attn, W_proj, b_proj):
    B, T, C = x.shape
    HS = C // NH
    G = 3 * NH

    x2 = x.reshape(B * T, C)

    heads_per_step = 4
    bn1 = heads_per_step * HS
    ng1 = G // heads_per_step
    qkv = pl.pallas_call(
        functools.partial(_qkv_kernel, heads_per_step=heads_per_step, hs=HS),
        grid=(B, ng1),
        in_specs=[
            pl.BlockSpec((T, C), lambda b, j: (b, 0)),
            pl.BlockSpec((C, bn1), lambda b, j: (0, j)),
            pl.BlockSpec((1, bn1), lambda b, j: (0, j)),
        ],
        out_specs=pl.BlockSpec((1, heads_per_step, T, HS),
                               lambda b, j: (b, j, 0, 0)),
        out_shape=jax.ShapeDtypeStruct((B, G, T, HS), jnp.float32),
    )(x2, W_attn, b_attn.reshape(1, 3 * C))

    BQ = 512
    nq = T // BQ
    scale = 1.0 / math.sqrt(HS)
    y = pl.pallas_call(
        functools.partial(_attn_kernel, bq=BQ, nq=nq, scale=scale),
        grid=(B, NH),
        in_specs=[
            pl.BlockSpec((1, 1, T, HS), lambda b, h: (b, h, 0, 0)),
            pl.BlockSpec((1, 1, T, HS), lambda b, h: (b, NH + h, 0, 0)),
            pl.BlockSpec((1, 1, T, HS), lambda b, h: (b, 2 * NH + h, 0, 0)),
        ],
        out_specs=pl.BlockSpec((1, 1, T, HS), lambda b, h: (b, h, 0, 0)),
        out_shape=jax.ShapeDtypeStruct((B, NH, T, HS), jnp.float32),
    )(qkv, qkv, qkv)

    bn3 = 512
    nn3 = C // bn3
    out = pl.pallas_call(
        functools.partial(_proj_kernel, nh=NH, hs=HS),
        grid=(B, nn3),
        in_specs=[
            pl.BlockSpec((1, NH, T, HS), lambda b, j: (b, 0, 0, 0)),
            pl.BlockSpec((NH, HS, bn3), lambda b, j: (0, 0, j)),
            pl.BlockSpec((1, bn3), lambda b, j: (0, j)),
        ],
        out_specs=pl.BlockSpec((1, T, bn3), lambda b, j: (b, 0, j)),
        out_shape=jax.ShapeDtypeStruct((B, T, C), jnp.float32),
    )(y, W_proj.reshape(NH, HS, C), b_proj.reshape(1, C))

    return out


# speedup vs baseline: 2.3577x; 1.0838x over previous
import functools
import math

import jax
import jax.numpy as jnp
from jax.experimental import pallas as pl
from jax.experimental.pallas import tpu as pltpu

NH = 16


def _qkv_kernel(x_ref, w_ref, b_ref, o_ref, *, heads_per_step, hs):
    r = jnp.dot(x_ref[...], w_ref[...], preferred_element_type=jnp.float32)
    r = r + b_ref[...]
    for hh in range(heads_per_step):
        o_ref[0, hh] = r[:, hh * hs:(hh + 1) * hs]


def _attn_kernel(q_ref, k_ref, v_ref, o_ref, *, bq, nq, scale):
    hs = q_ref.shape[3]
    for qi in range(nq):
        q = q_ref[0, 0, qi * bq:(qi + 1) * bq, :]
        s_blocks = []
        m_acc = None
        rows = jax.lax.broadcasted_iota(jnp.int32, (bq, bq), 0)
        cols = jax.lax.broadcasted_iota(jnp.int32, (bq, bq), 1)
        for j in range(qi + 1):
            kj = k_ref[0, 0, j * bq:(j + 1) * bq, :]
            s = jax.lax.dot_general(q, kj, (((1,), (1,)), ((), ())),
                                    preferred_element_type=jnp.float32) * scale
            if j == qi:
                s = jnp.where(cols <= rows, s, -1e30)
            s_blocks.append(s)
            m_acc = s if m_acc is None else jnp.maximum(m_acc, s)
        m = jnp.max(m_acc, axis=1, keepdims=True)
        l_acc = jnp.zeros((bq, bq), dtype=jnp.float32)
        acc = jnp.zeros((bq, hs), dtype=jnp.float32)
        for j in range(qi + 1):
            p = jnp.exp(s_blocks[j] - m)
            l_acc = l_acc + p
            vj = v_ref[0, 0, j * bq:(j + 1) * bq, :]
            acc = acc + jnp.dot(p, vj, preferred_element_type=jnp.float32)
        l = jnp.sum(l_acc, axis=1, keepdims=True)
        o_ref[qi * bq:(qi + 1) * bq, :] = acc / l


def kernel(x, W_attn, b_attn, W_proj, b_proj):
    B, T, C = x.shape
    HS = C // NH
    G = 3 * NH

    x2 = x.reshape(B * T, C)

    heads_per_step = 4
    bn1 = heads_per_step * HS
    ng1 = G // heads_per_step
    qkv = pl.pallas_call(
        functools.partial(_qkv_kernel, heads_per_step=heads_per_step, hs=HS),
        grid=(B, ng1),
        in_specs=[
            pl.BlockSpec((T, C), lambda b, j: (b, 0)),
            pl.BlockSpec((C, bn1), lambda b, j: (0, j)),
            pl.BlockSpec((1, bn1), lambda b, j: (0, j)),
        ],
        out_specs=pl.BlockSpec((1, heads_per_step, T, HS),
                               lambda b, j: (b, j, 0, 0)),
        out_shape=jax.ShapeDtypeStruct((B, G, T, HS), jnp.float32),
    )(x2, W_attn, b_attn.reshape(1, 3 * C))

    BQ = 512
    nq = T // BQ
    scale = 1.0 / math.sqrt(HS)
    y = pl.pallas_call(
        functools.partial(_attn_kernel, bq=BQ, nq=nq, scale=scale),
        grid=(B, NH),
        in_specs=[
            pl.BlockSpec((1, 1, T, HS), lambda b, h: (b, h, 0, 0)),
            pl.BlockSpec((1, 1, T, HS), lambda b, h: (b, NH + h, 0, 0)),
            pl.BlockSpec((1, 1, T, HS), lambda b, h: (b, 2 * NH + h, 0, 0)),
        ],
        out_specs=pl.BlockSpec((T, HS), lambda b, h: (b, h)),
        out_shape=jax.ShapeDtypeStruct((B * T, C), jnp.float32),
    )(qkv, qkv, qkv)

    bn3 = 512
    nn3 = C // bn3
    out = pl.pallas_call(
        functools.partial(_qkv_kernel, heads_per_step=1, hs=bn3),
        grid=(B, nn3),
        in_specs=[
            pl.BlockSpec((T, C), lambda b, j: (b, 0)),
            pl.BlockSpec((C, bn3), lambda b, j: (0, j)),
            pl.BlockSpec((1, bn3), lambda b, j: (0, j)),
        ],
        out_specs=pl.BlockSpec((1, 1, T, bn3), lambda b, j: (b, 0, 0, j)),
        out_shape=jax.ShapeDtypeStruct((B, 1, T, C), jnp.float32),
    )(y, W_proj, b_proj.reshape(1, C))

    return out.reshape(B, T, C)


# speedup vs baseline: 2.4316x; 1.0314x over previous
import functools
import math

import jax
import jax.numpy as jnp
from jax.experimental import pallas as pl
from jax.experimental.pallas import tpu as pltpu

NH = 16


def _qkv_kernel(x_ref, w_ref, b_ref, o_ref, *, heads_per_step, hs):
    r = jnp.dot(x_ref[...], w_ref[...], preferred_element_type=jnp.float32)
    r = r + b_ref[...]
    for hh in range(heads_per_step):
        o_ref[0, hh] = r[:, hh * hs:(hh + 1) * hs]


def _attn_kernel(q_ref, k_ref, v_ref, o_ref, *, bq, nq, scale):
    hs = q_ref.shape[3]
    rows = jax.lax.broadcasted_iota(jnp.int32, (bq, bq), 0)
    cols = jax.lax.broadcasted_iota(jnp.int32, (bq, bq), 1)
    log2e_scale = scale * 1.4426950408889634
    for qi in range(nq):
        q = q_ref[0, 0, qi * bq:(qi + 1) * bq, :] * log2e_scale
        s_blocks = []
        m_acc = None
        for j in range(qi + 1):
            kj = k_ref[0, 0, j * bq:(j + 1) * bq, :]
            s = jax.lax.dot_general(q, kj, (((1,), (1,)), ((), ())),
                                    preferred_element_type=jnp.float32)
            if j == qi:
                s = jnp.where(cols <= rows, s, -1e30)
            s_blocks.append(s)
            m_acc = s if m_acc is None else jnp.maximum(m_acc, s)
        m = jnp.max(m_acc, axis=1, keepdims=True)
        l_acc = jnp.zeros((bq, bq), dtype=jnp.float32)
        acc = jnp.zeros((bq, hs), dtype=jnp.float32)
        for j in range(qi + 1):
            p = jnp.exp2(s_blocks[j] - m)
            l_acc = l_acc + p
            vj = v_ref[0, 0, j * bq:(j + 1) * bq, :]
            acc = acc + jnp.dot(p, vj, preferred_element_type=jnp.float32)
        l = jnp.sum(l_acc, axis=1, keepdims=True)
        o_ref[qi * bq:(qi + 1) * bq, :] = acc * (1.0 / l)


def kernel(x, W_attn, b_attn, W_proj, b_proj):
    B, T, C = x.shape
    HS = C // NH
    G = 3 * NH

    x2 = x.reshape(B * T, C)

    heads_per_step = 4
    bn1 = heads_per_step * HS
    ng1 = G // heads_per_step
    qkv = pl.pallas_call(
        functools.partial(_qkv_kernel, heads_per_step=heads_per_step, hs=HS),
        grid=(B, ng1),
        in_specs=[
            pl.BlockSpec((T, C), lambda b, j: (b, 0)),
            pl.BlockSpec((C, bn1), lambda b, j: (0, j)),
            pl.BlockSpec((1, bn1), lambda b, j: (0, j)),
        ],
        out_specs=pl.BlockSpec((1, heads_per_step, T, HS),
                               lambda b, j: (b, j, 0, 0)),
        out_shape=jax.ShapeDtypeStruct((B, G, T, HS), jnp.float32),
    )(x2, W_attn, b_attn.reshape(1, 3 * C))

    BQ = 512
    nq = T // BQ
    scale = 1.0 / math.sqrt(HS)
    y = pl.pallas_call(
        functools.partial(_attn_kernel, bq=BQ, nq=nq, scale=scale),
        grid=(B, NH),
        in_specs=[
            pl.BlockSpec((1, 1, T, HS), lambda b, h: (b, h, 0, 0)),
            pl.BlockSpec((1, 1, T, HS), lambda b, h: (b, NH + h, 0, 0)),
            pl.BlockSpec((1, 1, T, HS), lambda b, h: (b, 2 * NH + h, 0, 0)),
        ],
        out_specs=pl.BlockSpec((T, HS), lambda b, h: (b, h)),
        out_shape=jax.ShapeDtypeStruct((B * T, C), jnp.float32),
    )(qkv, qkv, qkv)

    bn3 = 512
    nn3 = C // bn3
    out = pl.pallas_call(
        functools.partial(_qkv_kernel, heads_per_step=1, hs=bn3),
        grid=(B, nn3),
        in_specs=[
            pl.BlockSpec((T, C), lambda b, j: (b, 0)),
            pl.BlockSpec((C, bn3), lambda b, j: (0, j)),
            pl.BlockSpec((1, bn3), lambda b, j: (0, j)),
        ],
        out_specs=pl.BlockSpec((1, 1, T, bn3), lambda b, j: (b, 0, 0, j)),
        out_shape=jax.ShapeDtypeStruct((B, 1, T, C), jnp.float32),
    )(y, W_proj, b_proj.reshape(1, C))

    return out.reshape(B, T, C)
